# baseline (speedup 1.0000x reference)
import jax
import jax.numpy as jnp
from jax.experimental import pallas as pl
from jax.experimental.pallas import tpu as pltpu

_G = 512
_WP = 10
_WF = 10
_B = 32
_L = 110
_A = 110


_BPB = 4


def _edge_att_kernel(lens_ref, nf_ref, w_ref, alpha_ref):
    i0 = pl.program_id(0)
    nf = nf_ref[...]
    w = w_ref[...]
    nfh = nf.astype(jnp.bfloat16)
    wh = w.astype(jnp.bfloat16)
    att = jax.lax.dot_general(
        nfh, wh, (((2,), (1,)), ((), ())), preferred_element_type=jnp.float32)
    s = jax.lax.dot_general(
        nfh, att.astype(jnp.bfloat16), (((2,), (2,)), ((0,), (0,))),
        preferred_element_type=jnp.float32)

    row = jax.lax.broadcasted_iota(jnp.int32, (_L, _A), 0)
    col = jax.lax.broadcasted_iota(jnp.int32, (_L, _A), 1)
    band = (col >= row - _WP) & (col <= row + _WF)
    mc = jnp.maximum(row, col)
    for bb in range(_BPB):
        ln = lens_ref[i0 * _BPB + bb]
        active = band & (mc <= ln - 1)
        sb = jnp.where(active, s[bb], -1e9)
        e = jnp.exp(sb)
        denom = jnp.sum(e, axis=1, keepdims=True)
        inv = jnp.where(denom > 0.0, 1.0 / denom, 0.0)
        alpha_ref[bb] = e * inv


def kernel(node_features, text_len_tensor, edge_ind, weight):
    del edge_ind
    lens = text_len_tensor.astype(jnp.int32)
    grid_spec = pltpu.PrefetchScalarGridSpec(
        num_scalar_prefetch=1,
        grid=(_B // _BPB,),
        in_specs=[
            pl.BlockSpec((_BPB, _L, _G), lambda b, lens_ref: (b, 0, 0)),
            pl.BlockSpec((_G, _G), lambda b, lens_ref: (0, 0)),
        ],
        out_specs=pl.BlockSpec((_BPB, _L, _A), lambda b, lens_ref: (b, 0, 0)),
    )
    return pl.pallas_call(
        _edge_att_kernel,
        grid_spec=grid_spec,
        out_shape=jax.ShapeDtypeStruct((_B, _L, _A), jnp.float32),
    )(lens, node_features, weight)


# speedup vs baseline: 1.1049x; 1.1049x over previous
import jax
import jax.numpy as jnp
from jax.experimental import pallas as pl
from jax.experimental.pallas import tpu as pltpu

_G = 512
_WP = 10
_WF = 10
_B = 32
_L = 110
_A = 110


_BPB = 8


def _edge_att_kernel(lens_ref, nf_ref, w_ref, alpha_ref):
    i0 = pl.program_id(0)
    nf = nf_ref[...]
    w = w_ref[...]
    nfh = nf.astype(jnp.bfloat16)
    wh = w.astype(jnp.bfloat16)
    att = jax.lax.dot_general(
        nfh, wh, (((2,), (1,)), ((), ())), preferred_element_type=jnp.float32)
    s = jax.lax.dot_general(
        nfh, att.astype(jnp.bfloat16), (((2,), (2,)), ((0,), (0,))),
        preferred_element_type=jnp.float32)

    row = jax.lax.broadcasted_iota(jnp.int32, (_L, _A), 0)
    col = jax.lax.broadcasted_iota(jnp.int32, (_L, _A), 1)
    band = (col >= row - _WP) & (col <= row + _WF)
    mc = jnp.maximum(row, col)
    for bb in range(_BPB):
        ln = lens_ref[i0 * _BPB + bb]
        active = band & (mc <= ln - 1)
        sb = jnp.where(active, s[bb], -1e9)
        e = jnp.exp(sb)
        denom = jnp.sum(e, axis=1, keepdims=True)
        inv = jnp.where(denom > 0.0, 1.0 / denom, 0.0)
        alpha_ref[bb] = e * inv


def kernel(node_features, text_len_tensor, edge_ind, weight):
    del edge_ind
    lens = text_len_tensor.astype(jnp.int32)
    grid_spec = pltpu.PrefetchScalarGridSpec(
        num_scalar_prefetch=1,
        grid=(_B // _BPB,),
        in_specs=[
            pl.BlockSpec((_BPB, _L, _G), lambda b, lens_ref: (b, 0, 0)),
            pl.BlockSpec((_G, _G), lambda b, lens_ref: (0, 0)),
        ],
        out_specs=pl.BlockSpec((_BPB, _L, _A), lambda b, lens_ref: (b, 0, 0)),
    )
    return pl.pallas_call(
        _edge_att_kernel,
        grid_spec=grid_spec,
        out_shape=jax.ShapeDtypeStruct((_B, _L, _A), jnp.float32),
    )(lens, node_features, weight)
